# baseline (speedup 1.0000x reference)
import functools

import jax
import jax.numpy as jnp
from jax import lax
from jax.experimental import pallas as pl
from jax.experimental.pallas import tpu as pltpu
from jax.experimental.pallas import tpu_sc as plsc

N_NODES = 10000
F = 64
N_GRAPHS = 64
E_RAW = 320000

NC = 2
NS = 16
NW = NC * NS
CHUNK = 128
N_CHUNKS = E_RAW // CHUNK
RING = 78
EXTRA0 = NW * RING
N_PAD = 10240
STRIPE = N_PAD // NS
DEG_W = 16
NBUF = 3
ROWS_BLK = 1280

_HIGH = lax.Precision.HIGHEST


def _mesh():
    return plsc.VectorSubcoreMesh(core_axis_name="c", subcore_axis_name="s")


_SC_PARAMS = pltpu.CompilerParams(use_tc_tiling_on_sc=False)


def _load_idx(idx_hbm, idx_v, wid):
    pltpu.sync_copy(idx_hbm.at[pl.ds(wid * RING, RING)], idx_v.at[pl.ds(0, RING)])

    @pl.when(wid < N_CHUNKS - EXTRA0)
    def _extra():
        pltpu.sync_copy(idx_hbm.at[pl.ds(EXTRA0 + wid, 1)], idx_v.at[pl.ds(RING, 1)])


def _sc_degree(edges3):
    @functools.partial(
        pl.kernel,
        out_type=jax.ShapeDtypeStruct((N_PAD, 2 * F), jnp.float32),
        mesh=_mesh(),
        scratch_types=[
            pltpu.VMEM((RING + 1, 2, CHUNK), jnp.int32),
            pltpu.VMEM((CHUNK, DEG_W), jnp.float32),
            pltpu.VMEM_SHARED((N_PAD, DEG_W), jnp.float32),
            pltpu.SemaphoreType.DMA,
        ],
        compiler_params=_SC_PARAMS,
    )
    def deg_kernel(dst_hbm, out_hbm, dst_v, buf_v, acc_sh, sem):
        cid = lax.axis_index("c")
        sid = lax.axis_index("s")
        wid = sid * NC + cid

        @pl.loop(0, CHUNK)
        def _zero_row(r):
            buf_v[r, pl.ds(0, DEG_W)] = jnp.zeros((DEG_W,), jnp.float32)

        @pl.loop(0, STRIPE // CHUNK)
        def _zero_stripe(i):
            pltpu.sync_copy(buf_v, acc_sh.at[pl.ds(sid * STRIPE + i * CHUNK, CHUNK)])

        @pl.loop(0, CHUNK)
        def _ones_row(r):
            buf_v[r, pl.ds(0, DEG_W)] = jnp.ones((DEG_W,), jnp.float32)

        _load_idx(dst_hbm, dst_v, wid)
        plsc.subcore_barrier()

        @pl.loop(0, RING)
        def _chunk(c):
            pltpu.sync_copy(buf_v, acc_sh.at[dst_v.at[c, 1]], add=True)

        @pl.when(wid < N_CHUNKS - EXTRA0)
        def _extra():
            pltpu.sync_copy(buf_v, acc_sh.at[dst_v.at[RING, 1]], add=True)

        plsc.subcore_barrier()
        pltpu.sync_copy(
            acc_sh.at[pl.ds(sid * STRIPE, STRIPE)],
            out_hbm.at[pl.ds(sid * STRIPE, STRIPE), pl.ds(cid * F, DEG_W)],
        )

    return deg_kernel(edges3)


def _sc_aggregate(y, edges3):
    @functools.partial(
        pl.kernel,
        out_type=jax.ShapeDtypeStruct((N_PAD, 2 * F), jnp.float32),
        mesh=_mesh(),
        scratch_types=[
            pltpu.VMEM((RING + 1, 2, CHUNK), jnp.int32),
            pltpu.VMEM((CHUNK, F), jnp.float32),
            pltpu.VMEM((CHUNK, F), jnp.float32),
            pltpu.VMEM((CHUNK, F), jnp.float32),
            pltpu.VMEM_SHARED((N_PAD, F), jnp.float32),
            pltpu.VMEM_SHARED((N_PAD, F), jnp.float32),
            pltpu.SemaphoreType.DMA,
            pltpu.SemaphoreType.DMA,
            pltpu.SemaphoreType.DMA,
            pltpu.SemaphoreType.DMA,
        ],
        compiler_params=_SC_PARAMS,
    )
    def agg_kernel(y_hbm, e_hbm, out_hbm, ei_v,
                   r0, r1, r2, acc_sh, ytab_sh,
                   g0, g1, g2, ssem):
        cid = lax.axis_index("c")
        sid = lax.axis_index("s")
        wid = sid * NC + cid
        rows = (r0, r1, r2)
        gsem = (g0, g1, g2)
        groups = RING // NBUF

        @pl.loop(0, CHUNK)
        def _zero_row(r):
            @pl.loop(0, F, step=16)
            def _zero_lane(c0):
                r0[r, pl.ds(c0, 16)] = jnp.zeros((16,), jnp.float32)

        @pl.loop(0, STRIPE // CHUNK)
        def _zero_stripe(i):
            pltpu.sync_copy(r0, acc_sh.at[pl.ds(sid * STRIPE + i * CHUNK, CHUNK)])

        _load_idx(e_hbm, ei_v, wid)
        pltpu.sync_copy(y_hbm.at[pl.ds(sid * STRIPE, STRIPE), pl.ds(0, F)],
                        ytab_sh.at[pl.ds(sid * STRIPE, STRIPE)])
        plsc.subcore_barrier()

        for b in range(NBUF):
            pltpu.async_copy(ytab_sh.at[ei_v.at[b, 0]], rows[b], gsem[b])

        @pl.loop(0, groups)
        def _grp(g):
            c0 = g * NBUF
            for b in range(NBUF):
                c = c0 + b
                pltpu.make_async_copy(ytab_sh.at[ei_v.at[c, 0]], rows[b], gsem[b]).wait()
                pltpu.async_copy(rows[b], acc_sh.at[ei_v.at[c, 1]], ssem, add=True)

            @pl.when(g < groups - 1)
            def _refill():
                for b in range(NBUF):
                    c = c0 + b
                    pltpu.make_async_copy(rows[b], acc_sh.at[ei_v.at[c, 1]], ssem).wait()
                    pltpu.async_copy(ytab_sh.at[ei_v.at[c + NBUF, 0]], rows[b], gsem[b])

        for b in range(NBUF):
            c = (groups - 1) * NBUF + b
            pltpu.make_async_copy(rows[b], acc_sh.at[ei_v.at[c, 1]], ssem).wait()

        @pl.when(wid < N_CHUNKS - EXTRA0)
        def _tail():
            pltpu.sync_copy(ytab_sh.at[ei_v.at[RING, 0]], r0)
            pltpu.sync_copy(r0, acc_sh.at[ei_v.at[RING, 1]], add=True)

        plsc.subcore_barrier()
        pltpu.sync_copy(
            acc_sh.at[pl.ds(sid * STRIPE, STRIPE)],
            out_hbm.at[pl.ds(sid * STRIPE, STRIPE), pl.ds(cid * F, F)],
        )

    return agg_kernel(y, edges3)


def _tc_mm1(xp, W1):
    def body(x_ref, w_ref, o_ref):
        o_ref[...] = jnp.dot(
            x_ref[...], w_ref[...],
            preferred_element_type=jnp.float32, precision=_HIGH,
        )

    return pl.pallas_call(
        body,
        grid=(N_PAD // ROWS_BLK,),
        in_specs=[
            pl.BlockSpec((ROWS_BLK, 128), lambda i: (i, 0)),
            pl.BlockSpec((128, F), lambda i: (0, 0)),
        ],
        out_specs=pl.BlockSpec((ROWS_BLK, F), lambda i: (i, 0)),
        out_shape=jax.ShapeDtypeStruct((N_PAD, F), jnp.float32),
    )(xp, W1)


def _tc_prep(deg2, xw1):
    def body(deg_ref, xw_ref, dinv_ref, y_ref):
        deg = deg_ref[...]
        dtot = deg[:, 0:DEG_W] + deg[:, F:F + DEG_W] + 1.0
        dinv = lax.rsqrt(dtot)
        dinv_ref[...] = dinv
        y_ref[:, 0:F] = xw_ref[...] * dinv[:, 0:1]

    return pl.pallas_call(
        body,
        grid=(N_PAD // ROWS_BLK,),
        in_specs=[
            pl.BlockSpec((ROWS_BLK, 2 * F), lambda i: (i, 0)),
            pl.BlockSpec((ROWS_BLK, F), lambda i: (i, 0)),
        ],
        out_specs=(
            pl.BlockSpec((ROWS_BLK, DEG_W), lambda i: (i, 0)),
            pl.BlockSpec((ROWS_BLK, 2 * F), lambda i: (i, 0)),
        ),
        out_shape=(
            jax.ShapeDtypeStruct((N_PAD, DEG_W), jnp.float32),
            jax.ShapeDtypeStruct((N_PAD, 2 * F), jnp.float32),
        ),
    )(deg2, xw1)


def _tc_layer(agg, y, dinv16, b, Wn):
    def body(agg_ref, y_ref, dinv_ref, b_ref, w_ref, x_ref, ynext_ref):
        dinv = dinv_ref[...][:, 0:1]
        a = agg_ref[...]
        yv = y_ref[...][:, 0:F]
        x = jnp.maximum((a[:, 0:F] + a[:, F:] + yv) * dinv + b_ref[...], 0.0)
        x_ref[...] = x
        ynext_ref[:, 0:F] = dinv * jnp.dot(
            x, w_ref[...], preferred_element_type=jnp.float32, precision=_HIGH
        )

    return pl.pallas_call(
        body,
        grid=(N_PAD // ROWS_BLK,),
        in_specs=[
            pl.BlockSpec((ROWS_BLK, 2 * F), lambda i: (i, 0)),
            pl.BlockSpec((ROWS_BLK, 2 * F), lambda i: (i, 0)),
            pl.BlockSpec((ROWS_BLK, DEG_W), lambda i: (i, 0)),
            pl.BlockSpec((1, F), lambda i: (0, 0)),
            pl.BlockSpec((F, F), lambda i: (0, 0)),
        ],
        out_specs=(
            pl.BlockSpec((ROWS_BLK, F), lambda i: (i, 0)),
            pl.BlockSpec((ROWS_BLK, 2 * F), lambda i: (i, 0)),
        ),
        out_shape=(
            jax.ShapeDtypeStruct((N_PAD, F), jnp.float32),
            jax.ShapeDtypeStruct((N_PAD, 2 * F), jnp.float32),
        ),
    )(agg, y, dinv16, b, Wn)


def _tc_final(agg, y, dinv16, b, x1, x2, segf, Wl, bl):
    nsteps = N_PAD // ROWS_BLK

    def body(agg_ref, y_ref, dinv_ref, b_ref, x1_ref, x2_ref, seg_ref,
             wl_ref, bl_ref, o_ref, sums_acc, counts_acc):
        i = pl.program_id(0)
        dinv = dinv_ref[...][:, 0:1]
        a = agg_ref[...]
        yv = y_ref[...][:, 0:F]
        x3 = jnp.maximum((a[:, 0:F] + a[:, F:] + yv) * dinv + b_ref[...], 0.0)
        xm = (x1_ref[...] + x2_ref[...] + x3) * (1.0 / 3.0)
        gids = lax.broadcasted_iota(jnp.int32, (ROWS_BLK, N_GRAPHS), 1)
        sel = (seg_ref[...] == gids).astype(jnp.float32)
        part = lax.dot_general(
            sel, xm, (((0,), (0,)), ((), ())),
            preferred_element_type=jnp.float32, precision=_HIGH,
        )
        pcnt = jnp.sum(sel, axis=0, keepdims=True)

        @pl.when(i == 0)
        def _init():
            sums_acc[...] = part
            counts_acc[...] = pcnt

        @pl.when(i > 0)
        def _accum():
            sums_acc[...] += part
            counts_acc[...] += pcnt

        @pl.when(i == nsteps - 1)
        def _fin():
            pooled = sums_acc[...] / jnp.maximum(counts_acc[...], 1.0).reshape(
                N_GRAPHS, 1)
            logits = jnp.dot(
                pooled, wl_ref[...],
                preferred_element_type=jnp.float32, precision=_HIGH,
            ) + bl_ref[...]
            m = jnp.max(logits, axis=1, keepdims=True)
            e = jnp.exp(logits - m)
            o_ref[...] = e / jnp.sum(e, axis=1, keepdims=True)

    return pl.pallas_call(
        body,
        grid=(nsteps,),
        in_specs=[
            pl.BlockSpec((ROWS_BLK, 2 * F), lambda i: (i, 0)),
            pl.BlockSpec((ROWS_BLK, 2 * F), lambda i: (i, 0)),
            pl.BlockSpec((ROWS_BLK, DEG_W), lambda i: (i, 0)),
            pl.BlockSpec((1, F), lambda i: (0, 0)),
            pl.BlockSpec((ROWS_BLK, F), lambda i: (i, 0)),
            pl.BlockSpec((ROWS_BLK, F), lambda i: (i, 0)),
            pl.BlockSpec((ROWS_BLK, 1), lambda i: (i, 0)),
            pl.BlockSpec((F, 10), lambda i: (0, 0)),
            pl.BlockSpec((1, 10), lambda i: (0, 0)),
        ],
        out_specs=pl.BlockSpec((N_GRAPHS, 10), lambda i: (0, 0)),
        out_shape=jax.ShapeDtypeStruct((N_GRAPHS, 10), jnp.float32),
        scratch_shapes=[
            pltpu.VMEM((N_GRAPHS, N_GRAPHS), jnp.float32),
            pltpu.VMEM((1, N_GRAPHS), jnp.float32),
        ],
    )(agg, y, dinv16, b, x1, x2, segf, Wl, bl)


def kernel(X, L, batch, W1, b1, W2, b2, W3, b3, Wl, bl):
    x = X[0]
    seg = batch[0].astype(jnp.int32)

    xp = jnp.pad(x, ((0, N_PAD - N_NODES), (0, 0)))
    edges3 = jnp.transpose(
        L[0].astype(jnp.int32).reshape(2, N_CHUNKS, CHUNK), (1, 0, 2))
    segf = jnp.concatenate(
        [seg, jnp.full((N_PAD - N_NODES,), -1, jnp.int32)]
    ).reshape(N_PAD, 1)

    deg2 = _sc_degree(edges3)
    xw1 = _tc_mm1(xp, W1)
    dinv16, y1 = _tc_prep(deg2, xw1)

    agg1 = _sc_aggregate(y1, edges3)
    x1, y2 = _tc_layer(agg1, y1, dinv16, b1.reshape(1, F), W2)
    agg2 = _sc_aggregate(y2, edges3)
    x2, y3 = _tc_layer(agg2, y2, dinv16, b2.reshape(1, F), W3)
    agg3 = _sc_aggregate(y3, edges3)
    return _tc_final(agg3, y3, dinv16, b3.reshape(1, F), x1, x2, segf,
                     Wl, bl.reshape(1, 10))


# speedup vs baseline: 1.0596x; 1.0596x over previous
import functools

import jax
import jax.numpy as jnp
from jax import lax
from jax.experimental import pallas as pl
from jax.experimental.pallas import tpu as pltpu
from jax.experimental.pallas import tpu_sc as plsc

N_NODES = 10000
F = 64
N_GRAPHS = 64
E_RAW = 320000

NC = 2
NS = 16
NW = NC * NS
CHUNK = 128
N_CHUNKS = E_RAW // CHUNK
RING = 78
EXTRA0 = NW * RING
N_PAD = 10240
STRIPE = N_PAD // NS
DEG_W = 16
NBUF = 3
ROWS_BLK = 2560

_HIGH = lax.Precision.HIGHEST


def _mesh():
    return plsc.VectorSubcoreMesh(core_axis_name="c", subcore_axis_name="s")


_SC_PARAMS = pltpu.CompilerParams(use_tc_tiling_on_sc=False)


def _load_idx(idx_hbm, idx_v, wid):
    pltpu.sync_copy(idx_hbm.at[pl.ds(wid * RING, RING)], idx_v.at[pl.ds(0, RING)])

    @pl.when(wid < N_CHUNKS - EXTRA0)
    def _extra():
        pltpu.sync_copy(idx_hbm.at[pl.ds(EXTRA0 + wid, 1)], idx_v.at[pl.ds(RING, 1)])


def _sc_degree(edges3):
    @functools.partial(
        pl.kernel,
        out_type=jax.ShapeDtypeStruct((N_PAD, 2 * F), jnp.float32),
        mesh=_mesh(),
        scratch_types=[
            pltpu.VMEM((RING + 1, 2, CHUNK), jnp.int32),
            pltpu.VMEM((CHUNK, DEG_W), jnp.float32),
            pltpu.VMEM_SHARED((N_PAD, DEG_W), jnp.float32),
            pltpu.SemaphoreType.DMA,
        ],
        compiler_params=_SC_PARAMS,
    )
    def deg_kernel(dst_hbm, out_hbm, dst_v, buf_v, acc_sh, sem):
        cid = lax.axis_index("c")
        sid = lax.axis_index("s")
        wid = sid * NC + cid

        @pl.loop(0, CHUNK)
        def _zero_row(r):
            buf_v[r, pl.ds(0, DEG_W)] = jnp.zeros((DEG_W,), jnp.float32)

        @pl.loop(0, STRIPE // CHUNK)
        def _zero_stripe(i):
            pltpu.sync_copy(buf_v, acc_sh.at[pl.ds(sid * STRIPE + i * CHUNK, CHUNK)])

        @pl.loop(0, CHUNK)
        def _ones_row(r):
            buf_v[r, pl.ds(0, DEG_W)] = jnp.ones((DEG_W,), jnp.float32)

        _load_idx(dst_hbm, dst_v, wid)
        plsc.subcore_barrier()

        @pl.loop(0, RING)
        def _chunk(c):
            pltpu.sync_copy(buf_v, acc_sh.at[dst_v.at[c, 1]], add=True)

        @pl.when(wid < N_CHUNKS - EXTRA0)
        def _extra():
            pltpu.sync_copy(buf_v, acc_sh.at[dst_v.at[RING, 1]], add=True)

        plsc.subcore_barrier()
        pltpu.sync_copy(
            acc_sh.at[pl.ds(sid * STRIPE, STRIPE)],
            out_hbm.at[pl.ds(sid * STRIPE, STRIPE), pl.ds(cid * F, DEG_W)],
        )

    return deg_kernel(edges3)


def _sc_aggregate(y, edges3):
    @functools.partial(
        pl.kernel,
        out_type=jax.ShapeDtypeStruct((N_PAD, 2 * F), jnp.float32),
        mesh=_mesh(),
        scratch_types=[
            pltpu.VMEM((RING + 1, 2, CHUNK), jnp.int32),
            pltpu.VMEM((CHUNK, F), jnp.float32),
            pltpu.VMEM((CHUNK, F), jnp.float32),
            pltpu.VMEM((CHUNK, F), jnp.float32),
            pltpu.VMEM_SHARED((N_PAD, F), jnp.float32),
            pltpu.VMEM_SHARED((N_PAD, F), jnp.float32),
            pltpu.SemaphoreType.DMA,
            pltpu.SemaphoreType.DMA,
            pltpu.SemaphoreType.DMA,
            pltpu.SemaphoreType.DMA,
        ],
        compiler_params=_SC_PARAMS,
    )
    def agg_kernel(y_hbm, e_hbm, out_hbm, ei_v,
                   r0, r1, r2, acc_sh, ytab_sh,
                   g0, g1, g2, ssem):
        cid = lax.axis_index("c")
        sid = lax.axis_index("s")
        wid = sid * NC + cid
        rows = (r0, r1, r2)
        gsem = (g0, g1, g2)
        groups = RING // NBUF

        pltpu.async_copy(e_hbm.at[pl.ds(wid * RING, RING)],
                         ei_v.at[pl.ds(0, RING)], g0)
        pltpu.async_copy(y_hbm.at[pl.ds(sid * STRIPE, STRIPE), pl.ds(0, F)],
                         ytab_sh.at[pl.ds(sid * STRIPE, STRIPE)], g1)

        @pl.loop(0, CHUNK)
        def _zero_row(r):
            @pl.loop(0, F, step=16)
            def _zero_lane(c0):
                r0[r, pl.ds(c0, 16)] = jnp.zeros((16,), jnp.float32)

        @pl.loop(0, STRIPE // CHUNK)
        def _zero_stripe(i):
            pltpu.async_copy(r0, acc_sh.at[pl.ds(sid * STRIPE + i * CHUNK, CHUNK)],
                             ssem)

        @pl.loop(0, STRIPE // CHUNK)
        def _zero_drain(i):
            pltpu.make_async_copy(
                r0, acc_sh.at[pl.ds(sid * STRIPE + i * CHUNK, CHUNK)], ssem).wait()

        pltpu.make_async_copy(e_hbm.at[pl.ds(wid * RING, RING)],
                              ei_v.at[pl.ds(0, RING)], g0).wait()

        @pl.when(wid < N_CHUNKS - EXTRA0)
        def _extra_idx():
            pltpu.sync_copy(e_hbm.at[pl.ds(EXTRA0 + wid, 1)],
                            ei_v.at[pl.ds(RING, 1)])

        pltpu.make_async_copy(y_hbm.at[pl.ds(sid * STRIPE, STRIPE), pl.ds(0, F)],
                              ytab_sh.at[pl.ds(sid * STRIPE, STRIPE)], g1).wait()
        plsc.subcore_barrier()

        for b in range(NBUF):
            pltpu.async_copy(ytab_sh.at[ei_v.at[b, 0]], rows[b], gsem[b])

        @pl.loop(0, groups)
        def _grp(g):
            c0 = g * NBUF
            for b in range(NBUF):
                c = c0 + b
                pltpu.make_async_copy(ytab_sh.at[ei_v.at[c, 0]], rows[b], gsem[b]).wait()
                pltpu.async_copy(rows[b], acc_sh.at[ei_v.at[c, 1]], ssem, add=True)

            @pl.when(g < groups - 1)
            def _refill():
                for b in range(NBUF):
                    c = c0 + b
                    pltpu.make_async_copy(rows[b], acc_sh.at[ei_v.at[c, 1]], ssem).wait()
                    pltpu.async_copy(ytab_sh.at[ei_v.at[c + NBUF, 0]], rows[b], gsem[b])

        for b in range(NBUF):
            c = (groups - 1) * NBUF + b
            pltpu.make_async_copy(rows[b], acc_sh.at[ei_v.at[c, 1]], ssem).wait()

        @pl.when(wid < N_CHUNKS - EXTRA0)
        def _tail():
            pltpu.sync_copy(ytab_sh.at[ei_v.at[RING, 0]], r0)
            pltpu.sync_copy(r0, acc_sh.at[ei_v.at[RING, 1]], add=True)

        plsc.subcore_barrier()
        pltpu.sync_copy(
            acc_sh.at[pl.ds(sid * STRIPE, STRIPE)],
            out_hbm.at[pl.ds(sid * STRIPE, STRIPE), pl.ds(cid * F, F)],
        )

    return agg_kernel(y, edges3)


def _tc_mm1(xp, W1):
    def body(x_ref, w_ref, o_ref):
        o_ref[...] = jnp.dot(
            x_ref[...], w_ref[...],
            preferred_element_type=jnp.float32, precision=_HIGH,
        )

    return pl.pallas_call(
        body,
        grid=(N_PAD // ROWS_BLK,),
        in_specs=[
            pl.BlockSpec((ROWS_BLK, 128), lambda i: (i, 0)),
            pl.BlockSpec((128, F), lambda i: (0, 0)),
        ],
        out_specs=pl.BlockSpec((ROWS_BLK, F), lambda i: (i, 0)),
        out_shape=jax.ShapeDtypeStruct((N_PAD, F), jnp.float32),
    )(xp, W1)


def _tc_prep(deg2, xw1):
    def body(deg_ref, xw_ref, dinv_ref, y_ref):
        deg = deg_ref[...]
        dtot = deg[:, 0:DEG_W] + deg[:, F:F + DEG_W] + 1.0
        dinv = lax.rsqrt(dtot)
        dinv_ref[...] = dinv
        y_ref[:, 0:F] = xw_ref[...] * dinv[:, 0:1]

    return pl.pallas_call(
        body,
        grid=(N_PAD // ROWS_BLK,),
        in_specs=[
            pl.BlockSpec((ROWS_BLK, 2 * F), lambda i: (i, 0)),
            pl.BlockSpec((ROWS_BLK, F), lambda i: (i, 0)),
        ],
        out_specs=(
            pl.BlockSpec((ROWS_BLK, DEG_W), lambda i: (i, 0)),
            pl.BlockSpec((ROWS_BLK, 2 * F), lambda i: (i, 0)),
        ),
        out_shape=(
            jax.ShapeDtypeStruct((N_PAD, DEG_W), jnp.float32),
            jax.ShapeDtypeStruct((N_PAD, 2 * F), jnp.float32),
        ),
    )(deg2, xw1)


def _tc_layer(agg, y, dinv16, b, Wn):
    def body(agg_ref, y_ref, dinv_ref, b_ref, w_ref, x_ref, ynext_ref):
        dinv = dinv_ref[...][:, 0:1]
        a = agg_ref[...]
        yv = y_ref[...][:, 0:F]
        x = jnp.maximum((a[:, 0:F] + a[:, F:] + yv) * dinv + b_ref[...], 0.0)
        x_ref[...] = x
        ynext_ref[:, 0:F] = dinv * jnp.dot(
            x, w_ref[...], preferred_element_type=jnp.float32, precision=_HIGH
        )

    return pl.pallas_call(
        body,
        grid=(N_PAD // ROWS_BLK,),
        in_specs=[
            pl.BlockSpec((ROWS_BLK, 2 * F), lambda i: (i, 0)),
            pl.BlockSpec((ROWS_BLK, 2 * F), lambda i: (i, 0)),
            pl.BlockSpec((ROWS_BLK, DEG_W), lambda i: (i, 0)),
            pl.BlockSpec((1, F), lambda i: (0, 0)),
            pl.BlockSpec((F, F), lambda i: (0, 0)),
        ],
        out_specs=(
            pl.BlockSpec((ROWS_BLK, F), lambda i: (i, 0)),
            pl.BlockSpec((ROWS_BLK, 2 * F), lambda i: (i, 0)),
        ),
        out_shape=(
            jax.ShapeDtypeStruct((N_PAD, F), jnp.float32),
            jax.ShapeDtypeStruct((N_PAD, 2 * F), jnp.float32),
        ),
    )(agg, y, dinv16, b, Wn)


def _tc_final(agg, y, dinv16, b, x1, x2, segf, Wl, bl):
    nsteps = N_PAD // ROWS_BLK

    def body(agg_ref, y_ref, dinv_ref, b_ref, x1_ref, x2_ref, seg_ref,
             wl_ref, bl_ref, o_ref, sums_acc, counts_acc):
        i = pl.program_id(0)
        dinv = dinv_ref[...][:, 0:1]
        a = agg_ref[...]
        yv = y_ref[...][:, 0:F]
        x3 = jnp.maximum((a[:, 0:F] + a[:, F:] + yv) * dinv + b_ref[...], 0.0)
        xm = (x1_ref[...] + x2_ref[...] + x3) * (1.0 / 3.0)
        gids = lax.broadcasted_iota(jnp.int32, (ROWS_BLK, N_GRAPHS), 1)
        sel = (seg_ref[...] == gids).astype(jnp.float32)
        part = lax.dot_general(
            sel, xm, (((0,), (0,)), ((), ())),
            preferred_element_type=jnp.float32, precision=_HIGH,
        )
        pcnt = jnp.sum(sel, axis=0, keepdims=True)

        @pl.when(i == 0)
        def _init():
            sums_acc[...] = part
            counts_acc[...] = pcnt

        @pl.when(i > 0)
        def _accum():
            sums_acc[...] += part
            counts_acc[...] += pcnt

        @pl.when(i == nsteps - 1)
        def _fin():
            pooled = sums_acc[...] / jnp.maximum(counts_acc[...], 1.0).reshape(
                N_GRAPHS, 1)
            logits = jnp.dot(
                pooled, wl_ref[...],
                preferred_element_type=jnp.float32, precision=_HIGH,
            ) + bl_ref[...]
            m = jnp.max(logits, axis=1, keepdims=True)
            e = jnp.exp(logits - m)
            o_ref[...] = e / jnp.sum(e, axis=1, keepdims=True)

    return pl.pallas_call(
        body,
        grid=(nsteps,),
        in_specs=[
            pl.BlockSpec((ROWS_BLK, 2 * F), lambda i: (i, 0)),
            pl.BlockSpec((ROWS_BLK, 2 * F), lambda i: (i, 0)),
            pl.BlockSpec((ROWS_BLK, DEG_W), lambda i: (i, 0)),
            pl.BlockSpec((1, F), lambda i: (0, 0)),
            pl.BlockSpec((ROWS_BLK, F), lambda i: (i, 0)),
            pl.BlockSpec((ROWS_BLK, F), lambda i: (i, 0)),
            pl.BlockSpec((ROWS_BLK, 1), lambda i: (i, 0)),
            pl.BlockSpec((F, 10), lambda i: (0, 0)),
            pl.BlockSpec((1, 10), lambda i: (0, 0)),
        ],
        out_specs=pl.BlockSpec((N_GRAPHS, 10), lambda i: (0, 0)),
        out_shape=jax.ShapeDtypeStruct((N_GRAPHS, 10), jnp.float32),
        scratch_shapes=[
            pltpu.VMEM((N_GRAPHS, N_GRAPHS), jnp.float32),
            pltpu.VMEM((1, N_GRAPHS), jnp.float32),
        ],
    )(agg, y, dinv16, b, x1, x2, segf, Wl, bl)


def kernel(X, L, batch, W1, b1, W2, b2, W3, b3, Wl, bl):
    x = X[0]
    seg = batch[0].astype(jnp.int32)

    xp = jnp.pad(x, ((0, N_PAD - N_NODES), (0, 0)))
    edges3 = jnp.transpose(
        L[0].astype(jnp.int32).reshape(2, N_CHUNKS, CHUNK), (1, 0, 2))
    segf = jnp.concatenate(
        [seg, jnp.full((N_PAD - N_NODES,), -1, jnp.int32)]
    ).reshape(N_PAD, 1)

    deg2 = _sc_degree(edges3)
    xw1 = _tc_mm1(xp, W1)
    dinv16, y1 = _tc_prep(deg2, xw1)

    agg1 = _sc_aggregate(y1, edges3)
    x1, y2 = _tc_layer(agg1, y1, dinv16, b1.reshape(1, F), W2)
    agg2 = _sc_aggregate(y2, edges3)
    x2, y3 = _tc_layer(agg2, y2, dinv16, b2.reshape(1, F), W3)
    agg3 = _sc_aggregate(y3, edges3)
    return _tc_final(agg3, y3, dinv16, b3.reshape(1, F), x1, x2, segf,
                     Wl, bl.reshape(1, 10))


# speedup vs baseline: 1.0648x; 1.0049x over previous
import functools

import jax
import jax.numpy as jnp
from jax import lax
from jax.experimental import pallas as pl
from jax.experimental.pallas import tpu as pltpu
from jax.experimental.pallas import tpu_sc as plsc

N_NODES = 10000
F = 64
N_GRAPHS = 64
E_RAW = 320000

NC = 2
NS = 16
NW = NC * NS
CHUNK = 128
N_CHUNKS = E_RAW // CHUNK
RING = 78
EXTRA0 = NW * RING
N_PAD = 10240
STRIPE = N_PAD // NS
DEG_W = 16
NBUF = 3
ROWS_BLK = 2560

_HIGH = lax.Precision.HIGHEST


def _mesh():
    return plsc.VectorSubcoreMesh(core_axis_name="c", subcore_axis_name="s")


_SC_PARAMS = pltpu.CompilerParams(use_tc_tiling_on_sc=False)


def _load_idx(idx_hbm, idx_v, wid):
    pltpu.sync_copy(idx_hbm.at[pl.ds(wid * RING, RING)], idx_v.at[pl.ds(0, RING)])

    @pl.when(wid < N_CHUNKS - EXTRA0)
    def _extra():
        pltpu.sync_copy(idx_hbm.at[pl.ds(EXTRA0 + wid, 1)], idx_v.at[pl.ds(RING, 1)])


def _sc_degree(edges3):
    @functools.partial(
        pl.kernel,
        out_type=jax.ShapeDtypeStruct((N_PAD, 2 * F), jnp.float32),
        mesh=_mesh(),
        scratch_types=[
            pltpu.VMEM((RING + 1, 2, CHUNK), jnp.int32),
            pltpu.VMEM((CHUNK, DEG_W), jnp.float32),
            pltpu.VMEM_SHARED((N_PAD, DEG_W), jnp.float32),
            pltpu.SemaphoreType.DMA,
            pltpu.SemaphoreType.DMA,
        ],
        compiler_params=_SC_PARAMS,
    )
    def deg_kernel(dst_hbm, out_hbm, dst_v, buf_v, acc_sh, sem, isem):
        cid = lax.axis_index("c")
        sid = lax.axis_index("s")
        wid = sid * NC + cid

        pltpu.async_copy(dst_hbm.at[pl.ds(wid * RING, RING)],
                         dst_v.at[pl.ds(0, RING)], isem)

        @pl.loop(0, CHUNK)
        def _zero_row(r):
            buf_v[r, pl.ds(0, DEG_W)] = jnp.zeros((DEG_W,), jnp.float32)

        @pl.loop(0, STRIPE // CHUNK)
        def _zero_stripe(i):
            pltpu.async_copy(buf_v, acc_sh.at[pl.ds(sid * STRIPE + i * CHUNK, CHUNK)],
                             sem)

        @pl.loop(0, STRIPE // CHUNK)
        def _zero_drain(i):
            pltpu.make_async_copy(
                buf_v, acc_sh.at[pl.ds(sid * STRIPE + i * CHUNK, CHUNK)], sem).wait()

        @pl.loop(0, CHUNK)
        def _ones_row(r):
            buf_v[r, pl.ds(0, DEG_W)] = jnp.ones((DEG_W,), jnp.float32)

        pltpu.make_async_copy(dst_hbm.at[pl.ds(wid * RING, RING)],
                              dst_v.at[pl.ds(0, RING)], isem).wait()

        @pl.when(wid < N_CHUNKS - EXTRA0)
        def _extra_idx():
            pltpu.sync_copy(dst_hbm.at[pl.ds(EXTRA0 + wid, 1)],
                            dst_v.at[pl.ds(RING, 1)])
        plsc.subcore_barrier()

        @pl.loop(0, RING)
        def _chunk(c):
            pltpu.sync_copy(buf_v, acc_sh.at[dst_v.at[c, 1]], add=True)

        @pl.when(wid < N_CHUNKS - EXTRA0)
        def _extra():
            pltpu.sync_copy(buf_v, acc_sh.at[dst_v.at[RING, 1]], add=True)

        plsc.subcore_barrier()
        pltpu.sync_copy(
            acc_sh.at[pl.ds(sid * STRIPE, STRIPE)],
            out_hbm.at[pl.ds(sid * STRIPE, STRIPE), pl.ds(cid * F, DEG_W)],
        )

    return deg_kernel(edges3)


def _sc_aggregate(y, edges3):
    @functools.partial(
        pl.kernel,
        out_type=jax.ShapeDtypeStruct((N_PAD, 2 * F), jnp.float32),
        mesh=_mesh(),
        scratch_types=[
            pltpu.VMEM((RING + 1, 2, CHUNK), jnp.int32),
            pltpu.VMEM((CHUNK, F), jnp.float32),
            pltpu.VMEM((CHUNK, F), jnp.float32),
            pltpu.VMEM((CHUNK, F), jnp.float32),
            pltpu.VMEM_SHARED((N_PAD, F), jnp.float32),
            pltpu.VMEM_SHARED((N_PAD, F), jnp.float32),
            pltpu.SemaphoreType.DMA,
            pltpu.SemaphoreType.DMA,
            pltpu.SemaphoreType.DMA,
            pltpu.SemaphoreType.DMA,
        ],
        compiler_params=_SC_PARAMS,
    )
    def agg_kernel(y_hbm, e_hbm, out_hbm, ei_v,
                   r0, r1, r2, acc_sh, ytab_sh,
                   g0, g1, g2, ssem):
        cid = lax.axis_index("c")
        sid = lax.axis_index("s")
        wid = sid * NC + cid
        rows = (r0, r1, r2)
        gsem = (g0, g1, g2)
        groups = RING // NBUF

        pltpu.async_copy(e_hbm.at[pl.ds(wid * RING, RING)],
                         ei_v.at[pl.ds(0, RING)], g0)
        pltpu.async_copy(y_hbm.at[pl.ds(sid * STRIPE, STRIPE), pl.ds(0, F)],
                         ytab_sh.at[pl.ds(sid * STRIPE, STRIPE)], g1)

        @pl.loop(0, CHUNK)
        def _zero_row(r):
            @pl.loop(0, F, step=16)
            def _zero_lane(c0):
                r0[r, pl.ds(c0, 16)] = jnp.zeros((16,), jnp.float32)

        @pl.loop(0, STRIPE // CHUNK)
        def _zero_stripe(i):
            pltpu.async_copy(r0, acc_sh.at[pl.ds(sid * STRIPE + i * CHUNK, CHUNK)],
                             ssem)

        @pl.loop(0, STRIPE // CHUNK)
        def _zero_drain(i):
            pltpu.make_async_copy(
                r0, acc_sh.at[pl.ds(sid * STRIPE + i * CHUNK, CHUNK)], ssem).wait()

        pltpu.make_async_copy(e_hbm.at[pl.ds(wid * RING, RING)],
                              ei_v.at[pl.ds(0, RING)], g0).wait()

        @pl.when(wid < N_CHUNKS - EXTRA0)
        def _extra_idx():
            pltpu.sync_copy(e_hbm.at[pl.ds(EXTRA0 + wid, 1)],
                            ei_v.at[pl.ds(RING, 1)])

        pltpu.make_async_copy(y_hbm.at[pl.ds(sid * STRIPE, STRIPE), pl.ds(0, F)],
                              ytab_sh.at[pl.ds(sid * STRIPE, STRIPE)], g1).wait()
        plsc.subcore_barrier()

        for b in range(NBUF):
            pltpu.async_copy(ytab_sh.at[ei_v.at[b, 0]], rows[b], gsem[b])

        @pl.loop(0, groups)
        def _grp(g):
            c0 = g * NBUF
            for b in range(NBUF):
                c = c0 + b
                pltpu.make_async_copy(ytab_sh.at[ei_v.at[c, 0]], rows[b], gsem[b]).wait()
                pltpu.async_copy(rows[b], acc_sh.at[ei_v.at[c, 1]], ssem, add=True)

            @pl.when(g < groups - 1)
            def _refill():
                for b in range(NBUF):
                    c = c0 + b
                    pltpu.make_async_copy(rows[b], acc_sh.at[ei_v.at[c, 1]], ssem).wait()
                    pltpu.async_copy(ytab_sh.at[ei_v.at[c + NBUF, 0]], rows[b], gsem[b])

        for b in range(NBUF):
            c = (groups - 1) * NBUF + b
            pltpu.make_async_copy(rows[b], acc_sh.at[ei_v.at[c, 1]], ssem).wait()

        @pl.when(wid < N_CHUNKS - EXTRA0)
        def _tail():
            pltpu.sync_copy(ytab_sh.at[ei_v.at[RING, 0]], r0)
            pltpu.sync_copy(r0, acc_sh.at[ei_v.at[RING, 1]], add=True)

        plsc.subcore_barrier()
        pltpu.sync_copy(
            acc_sh.at[pl.ds(sid * STRIPE, STRIPE)],
            out_hbm.at[pl.ds(sid * STRIPE, STRIPE), pl.ds(cid * F, F)],
        )

    return agg_kernel(y, edges3)


def _tc_mm1(xp, W1):
    def body(x_ref, w_ref, o_ref):
        o_ref[...] = jnp.dot(
            x_ref[...], w_ref[...],
            preferred_element_type=jnp.float32, precision=_HIGH,
        )

    return pl.pallas_call(
        body,
        grid=(N_PAD // ROWS_BLK,),
        in_specs=[
            pl.BlockSpec((ROWS_BLK, 128), lambda i: (i, 0)),
            pl.BlockSpec((128, F), lambda i: (0, 0)),
        ],
        out_specs=pl.BlockSpec((ROWS_BLK, F), lambda i: (i, 0)),
        out_shape=jax.ShapeDtypeStruct((N_PAD, F), jnp.float32),
    )(xp, W1)


def _tc_prep(deg2, xw1):
    def body(deg_ref, xw_ref, dinv_ref, y_ref):
        deg = deg_ref[...]
        dtot = deg[:, 0:DEG_W] + deg[:, F:F + DEG_W] + 1.0
        dinv = lax.rsqrt(dtot)
        dinv_ref[...] = dinv
        y_ref[:, 0:F] = xw_ref[...] * dinv[:, 0:1]

    return pl.pallas_call(
        body,
        grid=(N_PAD // ROWS_BLK,),
        in_specs=[
            pl.BlockSpec((ROWS_BLK, 2 * F), lambda i: (i, 0)),
            pl.BlockSpec((ROWS_BLK, F), lambda i: (i, 0)),
        ],
        out_specs=(
            pl.BlockSpec((ROWS_BLK, DEG_W), lambda i: (i, 0)),
            pl.BlockSpec((ROWS_BLK, 2 * F), lambda i: (i, 0)),
        ),
        out_shape=(
            jax.ShapeDtypeStruct((N_PAD, DEG_W), jnp.float32),
            jax.ShapeDtypeStruct((N_PAD, 2 * F), jnp.float32),
        ),
    )(deg2, xw1)


def _tc_layer(agg, y, dinv16, b, Wn):
    def body(agg_ref, y_ref, dinv_ref, b_ref, w_ref, x_ref, ynext_ref):
        dinv = dinv_ref[...][:, 0:1]
        a = agg_ref[...]
        yv = y_ref[...][:, 0:F]
        x = jnp.maximum((a[:, 0:F] + a[:, F:] + yv) * dinv + b_ref[...], 0.0)
        x_ref[...] = x
        ynext_ref[:, 0:F] = dinv * jnp.dot(
            x, w_ref[...], preferred_element_type=jnp.float32, precision=_HIGH
        )

    return pl.pallas_call(
        body,
        grid=(N_PAD // ROWS_BLK,),
        in_specs=[
            pl.BlockSpec((ROWS_BLK, 2 * F), lambda i: (i, 0)),
            pl.BlockSpec((ROWS_BLK, 2 * F), lambda i: (i, 0)),
            pl.BlockSpec((ROWS_BLK, DEG_W), lambda i: (i, 0)),
            pl.BlockSpec((1, F), lambda i: (0, 0)),
            pl.BlockSpec((F, F), lambda i: (0, 0)),
        ],
        out_specs=(
            pl.BlockSpec((ROWS_BLK, F), lambda i: (i, 0)),
            pl.BlockSpec((ROWS_BLK, 2 * F), lambda i: (i, 0)),
        ),
        out_shape=(
            jax.ShapeDtypeStruct((N_PAD, F), jnp.float32),
            jax.ShapeDtypeStruct((N_PAD, 2 * F), jnp.float32),
        ),
    )(agg, y, dinv16, b, Wn)


def _tc_final(agg, y, dinv16, b, x1, x2, segf, Wl, bl):
    nsteps = N_PAD // ROWS_BLK

    def body(agg_ref, y_ref, dinv_ref, b_ref, x1_ref, x2_ref, seg_ref,
             wl_ref, bl_ref, o_ref, sums_acc, counts_acc):
        i = pl.program_id(0)
        dinv = dinv_ref[...][:, 0:1]
        a = agg_ref[...]
        yv = y_ref[...][:, 0:F]
        x3 = jnp.maximum((a[:, 0:F] + a[:, F:] + yv) * dinv + b_ref[...], 0.0)
        xm = (x1_ref[...] + x2_ref[...] + x3) * (1.0 / 3.0)
        gids = lax.broadcasted_iota(jnp.int32, (ROWS_BLK, N_GRAPHS), 1)
        sel = (seg_ref[...] == gids).astype(jnp.float32)
        part = lax.dot_general(
            sel, xm, (((0,), (0,)), ((), ())),
            preferred_element_type=jnp.float32, precision=_HIGH,
        )
        pcnt = jnp.sum(sel, axis=0, keepdims=True)

        @pl.when(i == 0)
        def _init():
            sums_acc[...] = part
            counts_acc[...] = pcnt

        @pl.when(i > 0)
        def _accum():
            sums_acc[...] += part
            counts_acc[...] += pcnt

        @pl.when(i == nsteps - 1)
        def _fin():
            pooled = sums_acc[...] / jnp.maximum(counts_acc[...], 1.0).reshape(
                N_GRAPHS, 1)
            logits = jnp.dot(
                pooled, wl_ref[...],
                preferred_element_type=jnp.float32, precision=_HIGH,
            ) + bl_ref[...]
            m = jnp.max(logits, axis=1, keepdims=True)
            e = jnp.exp(logits - m)
            o_ref[...] = e / jnp.sum(e, axis=1, keepdims=True)

    return pl.pallas_call(
        body,
        grid=(nsteps,),
        in_specs=[
            pl.BlockSpec((ROWS_BLK, 2 * F), lambda i: (i, 0)),
            pl.BlockSpec((ROWS_BLK, 2 * F), lambda i: (i, 0)),
            pl.BlockSpec((ROWS_BLK, DEG_W), lambda i: (i, 0)),
            pl.BlockSpec((1, F), lambda i: (0, 0)),
            pl.BlockSpec((ROWS_BLK, F), lambda i: (i, 0)),
            pl.BlockSpec((ROWS_BLK, F), lambda i: (i, 0)),
            pl.BlockSpec((ROWS_BLK, 1), lambda i: (i, 0)),
            pl.BlockSpec((F, 10), lambda i: (0, 0)),
            pl.BlockSpec((1, 10), lambda i: (0, 0)),
        ],
        out_specs=pl.BlockSpec((N_GRAPHS, 10), lambda i: (0, 0)),
        out_shape=jax.ShapeDtypeStruct((N_GRAPHS, 10), jnp.float32),
        scratch_shapes=[
            pltpu.VMEM((N_GRAPHS, N_GRAPHS), jnp.float32),
            pltpu.VMEM((1, N_GRAPHS), jnp.float32),
        ],
    )(agg, y, dinv16, b, x1, x2, segf, Wl, bl)


def kernel(X, L, batch, W1, b1, W2, b2, W3, b3, Wl, bl):
    x = X[0]
    seg = batch[0].astype(jnp.int32)

    xp = jnp.pad(x, ((0, N_PAD - N_NODES), (0, 0)))
    edges3 = jnp.transpose(
        L[0].astype(jnp.int32).reshape(2, N_CHUNKS, CHUNK), (1, 0, 2))
    segf = jnp.concatenate(
        [seg, jnp.full((N_PAD - N_NODES,), -1, jnp.int32)]
    ).reshape(N_PAD, 1)

    deg2 = _sc_degree(edges3)
    xw1 = _tc_mm1(xp, W1)
    dinv16, y1 = _tc_prep(deg2, xw1)

    agg1 = _sc_aggregate(y1, edges3)
    x1, y2 = _tc_layer(agg1, y1, dinv16, b1.reshape(1, F), W2)
    agg2 = _sc_aggregate(y2, edges3)
    x2, y3 = _tc_layer(agg2, y2, dinv16, b2.reshape(1, F), W3)
    agg3 = _sc_aggregate(y3, edges3)
    return _tc_final(agg3, y3, dinv16, b3.reshape(1, F), x1, x2, segf,
                     Wl, bl.reshape(1, 10))
